# baseline (speedup 1.0000x reference)
import jax
import jax.numpy as jnp
from jax import lax
from jax.experimental import pallas as pl
from jax.experimental.pallas import tpu as pltpu
from jax.experimental.pallas import tpu_sc as plsc

_N = 10000
_E = 320000
_F = 50
_H1 = 4
_C1 = 256
_HID = 1024
_H3 = 6
_NC = 121
_K = 128
_SUB = 16
_NPW = 64
_NWIN = 160
_NPAD = _NWIN * _NPW
_ET = _E + _N
_ETP = ((_ET + _K - 1) // _K) * _K
_NWORK = 32
_WPW = _NWIN // _NWORK
_MBLK = 256
_GRID = _NPAD // _MBLK
_NEG = -3.0e38

_f32 = jnp.float32
_i32 = jnp.int32


_SC_CACHE = {}


def _gat_sc(h, asrc_f, adst_f, srcp, dstp, sched_f, zeros_f, H, C, NPW):
    key = (H, C, NPW)
    if key not in _SC_CACHE:
        _SC_CACHE[key] = _build_sc(H, C, NPW)
    return _SC_CACHE[key](srcp, dstp, sched_f, asrc_f, adst_f, h, zeros_f)


def _build_sc(H, C, NPW):
    D = _HID
    nwin = _NPAD // NPW
    wpw = nwin // _NWORK
    grp = _K // 16
    mesh = plsc.VectorSubcoreMesh(core_axis_name="c", subcore_axis_name="s",
                                  num_cores=2, num_subcores=16)

    def body(src_hbm, dst_hbm, sched_hbm, asrc_hbm, adst_hbm, h_hbm, zero_hbm,
             msg_hbm, den_hbm,
             src_c, dst_c, asrc_t, adslab, emax, dens, wbuf,
             rows, tmpf, tmpi, schedrow, acc, sem0, sem1):
        wid = lax.axis_index("s") * 2 + lax.axis_index("c")
        iota = lax.iota(_i32, 16)
        zf = jnp.zeros((16,), _f32)
        ninf = jnp.full((16,), _NEG, _f32)
        pltpu.sync_copy(asrc_hbm, asrc_t)

        def wbody(t, wcarry):
            win = wid * wpw + t
            d0 = win * NPW
            pltpu.sync_copy(sched_hbm.at[pl.ds(win * 16, 16)], schedrow)
            sv = schedrow[...]
            e0 = sv[0]
            e1 = sv[1]
            c0 = sv[2]
            nch = sv[3]
            for i in range(NPW * H // 16):
                emax[pl.ds(i * 16, 16)] = ninf
                dens[pl.ds(i * 16, 16)] = zf
            pltpu.sync_copy(adst_hbm.at[pl.ds(d0 * H, NPW * H)], adslab)
            pltpu.sync_copy(zero_hbm, acc)

            def prelude(g):
                dstv = dst_c[pl.ds(g * 16, 16)]
                dl = jnp.clip(dstv - d0, 0, NPW - 1)
                tmpi[...] = dl
                pdl = [plsc.load_gather(tmpi, [jnp.maximum(iota - d, 0)])
                       for d in (1, 2, 4, 8)]
                ndl = plsc.load_gather(tmpi, [jnp.minimum(iota + 1, 15)])
                last = (dl != ndl) | (iota == 15)
                return dl, pdl, last

            def escore(g, dl, valid, hcol):
                srcv = src_c[pl.ds(g * 16, 16)]
                ash = plsc.load_gather(asrc_t, [srcv * H + hcol])
                adh = plsc.load_gather(adslab, [dl * H + hcol])
                s = ash + adh
                ev = jnp.maximum(s, 0.2 * s)
                return jnp.where(valid, ev, _NEG)

            def p1(ci, carry):
                cbase = (c0 + ci) * _K
                pltpu.sync_copy(src_hbm.at[pl.ds(cbase, _K)], src_c)
                pltpu.sync_copy(dst_hbm.at[pl.ds(cbase, _K)], dst_c)
                for g in range(grp):
                    dl, pdl, last = prelude(g)
                    eidx = cbase + g * 16 + iota
                    valid = (eidx >= e0) & (eidx < e1)
                    for hcol in range(H):
                        m = escore(g, dl, valid, hcol)
                        for d, p in zip((1, 2, 4, 8), pdl):
                            tmpf[...] = m
                            prev = plsc.load_gather(
                                tmpf, [jnp.maximum(iota - d, 0)])
                            m = jnp.maximum(m, jnp.where(dl == p, prev, _NEG))
                        fi = dl * H + hcol
                        cur = plsc.load_gather(emax, [fi])
                        plsc.store_scatter(emax, [fi], jnp.maximum(cur, m),
                                           mask=last)
                return carry

            lax.fori_loop(0, nch, p1, 0)

            def p2(ci, carry):
                cbase = (c0 + ci) * _K
                pltpu.sync_copy(src_hbm.at[pl.ds(cbase, _K)], src_c)
                pltpu.sync_copy(dst_hbm.at[pl.ds(cbase, _K)], dst_c)
                for g in range(grp):
                    dl, pdl, last = prelude(g)
                    eidx = cbase + g * 16 + iota
                    valid = (eidx >= e0) & (eidx < e1)
                    for hcol in range(H):
                        ev = escore(g, dl, valid, hcol)
                        fi = dl * H + hcol
                        mx = plsc.load_gather(emax, [fi])
                        w = jnp.exp(ev - mx)
                        w = jnp.where(valid, w, 0.0)
                        plsc.store_scatter(wbuf, [(g * 16 + iota) * H + hcol],
                                           w)
                        m = w
                        for d, p in zip((1, 2, 4, 8), pdl):
                            tmpf[...] = m
                            prev = plsc.load_gather(
                                tmpf, [jnp.maximum(iota - d, 0)])
                            m = m + jnp.where((dl == p) & (iota >= d),
                                              prev, 0.0)
                        cur = plsc.load_gather(dens, [fi])
                        plsc.store_scatter(dens, [fi], cur + m, mask=last)
                sems = (sem0, sem1)
                cps = {0: pltpu.async_copy(
                    h_hbm.at[src_c[pl.ds(0, 16)]], rows.at[0], sem0)}
                for s in range(grp):
                    rb = s % 2
                    cps[s].wait()
                    if s + 1 < grp:
                        nb = (s + 1) % 2
                        cps[s + 1] = pltpu.async_copy(
                            h_hbm.at[src_c[pl.ds((s + 1) * 16, 16)]],
                            rows.at[nb], sems[nb])

                    def jbody(j, c2, s=s, rb=rb):
                        je = s * 16 + j
                        djv = plsc.load_gather(
                            dst_c, [jnp.full((16,), je, _i32)])
                        dj = jnp.clip(djv - d0, 0, NPW - 1)
                        dbase = dj * D
                        frb = jnp.full((16,), rb, _i32)
                        fj = jnp.full((16,), j, _i32)
                        for hcol in range(H):
                            wsp = plsc.load_gather(
                                wbuf,
                                [jnp.full((16,), je * H + hcol, _i32)])
                            for kk in range(C // 16):
                                off = hcol * C + kk * 16
                                v = plsc.load_gather(
                                    rows, [frb, fj, off + iota])
                                plsc.addupdate_scatter(
                                    acc, [dbase + (off + iota)], wsp * v)
                        return c2

                    lax.fori_loop(0, 16, jbody, 0)
                return carry

            lax.fori_loop(0, nch, p2, 0)
            pltpu.sync_copy(acc, msg_hbm.at[pl.ds(d0 * D, NPW * D)])
            pltpu.sync_copy(dens, den_hbm.at[pl.ds(d0 * H, NPW * H)])
            return wcarry

        lax.fori_loop(0, wpw, wbody, 0)

    kfn = pl.kernel(
        body,
        out_type=(jax.ShapeDtypeStruct((_NPAD * D,), _f32),
                  jax.ShapeDtypeStruct((_NPAD * H,), _f32)),
        mesh=mesh,
        compiler_params=pltpu.CompilerParams(needs_layout_passes=False),
        scratch_types=[
            pltpu.VMEM((_K,), _i32),
            pltpu.VMEM((_K,), _i32),
            pltpu.VMEM((_NPAD * H,), _f32),
            pltpu.VMEM((NPW * H,), _f32),
            pltpu.VMEM((NPW * H,), _f32),
            pltpu.VMEM((NPW * H,), _f32),
            pltpu.VMEM((_K * H,), _f32),
            pltpu.VMEM((2, 16, D), _f32),
            pltpu.VMEM((16,), _f32),
            pltpu.VMEM((16,), _i32),
            pltpu.VMEM((16,), _i32),
            pltpu.VMEM((NPW * D,), _f32),
            pltpu.SemaphoreType.DMA,
            pltpu.SemaphoreType.DMA,
        ],
    )
    return kfn


def _alpha_cols(hp, aflat, H, C):
    cols = [jnp.sum(hp[:, hd * C:(hd + 1) * C] * aflat[:, hd * C:(hd + 1) * C],
                    axis=1, keepdims=True) for hd in range(H)]
    return jnp.concatenate(cols, axis=1)


def _tc_first(x, W, Wl, bl, asf, adf):
    def body(x_ref, w_ref, wl_ref, bl_ref, asf_ref, adf_ref,
             hp_ref, as_ref, ad_ref, skip_ref):
        hp = jnp.dot(x_ref[...], w_ref[...], preferred_element_type=_f32)
        hp_ref[...] = hp
        as_ref[...] = _alpha_cols(hp, asf_ref[...], _H1, _C1)
        ad_ref[...] = _alpha_cols(hp, adf_ref[...], _H1, _C1)
        skip_ref[...] = jnp.dot(x_ref[...], wl_ref[...],
                                preferred_element_type=_f32) + bl_ref[...]

    return pl.pallas_call(
        body,
        grid=(_GRID,),
        in_specs=[
            pl.BlockSpec((_MBLK, _F), lambda i: (i, 0)),
            pl.BlockSpec((_F, _HID), lambda i: (0, 0)),
            pl.BlockSpec((_F, _HID), lambda i: (0, 0)),
            pl.BlockSpec((1, _HID), lambda i: (0, 0)),
            pl.BlockSpec((1, _HID), lambda i: (0, 0)),
            pl.BlockSpec((1, _HID), lambda i: (0, 0)),
        ],
        out_specs=[
            pl.BlockSpec((_MBLK, _HID), lambda i: (i, 0)),
            pl.BlockSpec((_MBLK, _H1), lambda i: (i, 0)),
            pl.BlockSpec((_MBLK, _H1), lambda i: (i, 0)),
            pl.BlockSpec((_MBLK, _HID), lambda i: (i, 0)),
        ],
        out_shape=[
            jax.ShapeDtypeStruct((_N, _HID), _f32),
            jax.ShapeDtypeStruct((_NPAD, _H1), _f32),
            jax.ShapeDtypeStruct((_NPAD, _H1), _f32),
            jax.ShapeDtypeStruct((_N, _HID), _f32),
        ],
    )(x, W, Wl, bl, asf, adf)


def _tc_mid(msg, den, skipp, b_row, W, Wl, bl_row, asf, adf, Ek, H, C, Dsk):
    def body(msg_ref, den_ref, skipp_ref, b_ref, w_ref, wl_ref, bl_ref,
             asf_ref, adf_ref, ek_ref,
             hp_ref, as_ref, ad_ref, skip_ref):
        denr = jnp.dot(den_ref[...], ek_ref[...], preferred_element_type=_f32)
        z = msg_ref[...] / (denr + 1e-16) + b_ref[...] + skipp_ref[...]
        z = jnp.where(z > 0, z, jnp.exp(z) - 1.0)
        hp = jnp.dot(z, w_ref[...], preferred_element_type=_f32)
        hp_ref[...] = hp
        as_ref[...] = _alpha_cols(hp, asf_ref[...], H, C)
        ad_ref[...] = _alpha_cols(hp, adf_ref[...], H, C)
        skip_ref[...] = jnp.dot(z, wl_ref[...],
                                preferred_element_type=_f32) + bl_ref[...]

    return pl.pallas_call(
        body,
        grid=(_GRID,),
        in_specs=[
            pl.BlockSpec((_MBLK, _HID), lambda i: (i, 0)),
            pl.BlockSpec((_MBLK, _H1), lambda i: (i, 0)),
            pl.BlockSpec((_MBLK, _HID), lambda i: (i, 0)),
            pl.BlockSpec((1, _HID), lambda i: (0, 0)),
            pl.BlockSpec((_HID, _HID), lambda i: (0, 0)),
            pl.BlockSpec((_HID, Dsk), lambda i: (0, 0)),
            pl.BlockSpec((1, Dsk), lambda i: (0, 0)),
            pl.BlockSpec((1, _HID), lambda i: (0, 0)),
            pl.BlockSpec((1, _HID), lambda i: (0, 0)),
            pl.BlockSpec((_H1, _HID), lambda i: (0, 0)),
        ],
        out_specs=[
            pl.BlockSpec((_MBLK, _HID), lambda i: (i, 0)),
            pl.BlockSpec((_MBLK, H), lambda i: (i, 0)),
            pl.BlockSpec((_MBLK, H), lambda i: (i, 0)),
            pl.BlockSpec((_MBLK, Dsk), lambda i: (i, 0)),
        ],
        out_shape=[
            jax.ShapeDtypeStruct((_N, _HID), _f32),
            jax.ShapeDtypeStruct((_NPAD, H), _f32),
            jax.ShapeDtypeStruct((_NPAD, H), _f32),
            jax.ShapeDtypeStruct((_N, Dsk), _f32),
        ],
    )(msg, den, skipp, b_row, W, Wl, bl_row, asf, adf, Ek)


def _tc_final(msg3, den3, skip3, b3_row):
    def body(msg_ref, den_ref, skip_ref, b_ref, out_ref):
        acc = None
        for hd in range(_H3):
            term = (msg_ref[:, hd * 128:hd * 128 + _NC]
                    / (den_ref[:, hd:hd + 1] + 1e-16))
            acc = term if acc is None else acc + term
        out_ref[...] = acc * (1.0 / _H3) + b_ref[...] + skip_ref[...]

    return pl.pallas_call(
        body,
        grid=(_GRID,),
        in_specs=[
            pl.BlockSpec((_MBLK, _HID), lambda i: (i, 0)),
            pl.BlockSpec((_MBLK, _H3), lambda i: (i, 0)),
            pl.BlockSpec((_MBLK, _NC), lambda i: (i, 0)),
            pl.BlockSpec((1, _NC), lambda i: (0, 0)),
        ],
        out_specs=pl.BlockSpec((_MBLK, _NC), lambda i: (i, 0)),
        out_shape=jax.ShapeDtypeStruct((_N, _NC), _f32),
    )(msg3, den3, skip3, b3_row)


def _edge_prep(edge_index):
    loops = jnp.arange(_N, dtype=_i32)
    src0 = jnp.concatenate([edge_index[0], loops])
    dst0 = jnp.concatenate([edge_index[1], loops])
    dst_s, src_s = lax.sort((dst0, src0), num_keys=1)
    pad = _ETP - _ET
    srcp = jnp.concatenate([src_s, jnp.zeros((pad,), _i32)])
    dstp = jnp.concatenate([dst_s, jnp.full((pad,), _NPAD - 1, _i32)])
    return srcp, dstp


def _sched_for(dstp, npw):
    nwin = _NPAD // npw
    bnd = jnp.searchsorted(
        dstp, jnp.arange(0, _NPAD + 1, npw, dtype=_i32),
        side='left').astype(_i32)
    e0 = bnd[:-1]
    e1 = bnd[1:]
    c0 = e0 // _K
    nch = jnp.maximum((e1 + _K - 1) // _K - c0, 0)
    sched = jnp.concatenate(
        [e0[:, None], e1[:, None], c0[:, None], nch[:, None],
         jnp.zeros((nwin, 12), _i32)], axis=1)
    return sched.reshape(-1)


def kernel(x, edge_index, W1, a_src1, a_dst1, b1, Wl1, bl1, W2, a_src2,
           a_dst2, b2, Wl2, bl2, W3, a_src3, a_dst3, b3, Wl3, bl3):
    srcp, dstp = _edge_prep(edge_index)
    sched32 = _sched_for(dstp, 32)
    sched16 = _sched_for(dstp, 16)
    zeros32 = jnp.zeros((32 * _HID,), _f32)
    zeros16 = jnp.zeros((16 * _HID,), _f32)
    Ek = jnp.repeat(jnp.eye(_H1, dtype=_f32), _C1, axis=1)

    hp1, as1, ad1, skip1 = _tc_first(
        x, W1, Wl1, bl1.reshape(1, _HID), a_src1.reshape(1, _HID),
        a_dst1.reshape(1, _HID))
    msg1, den1f = _gat_sc(hp1, as1.reshape(-1), ad1.reshape(-1), srcp, dstp,
                          sched32, zeros32, _H1, _C1, 32)
    hp2, as2, ad2, skip2 = _tc_mid(
        msg1.reshape(_NPAD, _HID), den1f.reshape(_NPAD, _H1), skip1,
        b1.reshape(1, _HID),
        W2, Wl2, bl2.reshape(1, _HID), a_src2.reshape(1, _HID),
        a_dst2.reshape(1, _HID), Ek, _H1, _C1, _HID)
    msg2, den2f = _gat_sc(hp2, as2.reshape(-1), ad2.reshape(-1), srcp, dstp,
                          sched32, zeros32, _H1, _C1, 32)
    W3p = jnp.pad(W3.reshape(_HID, _H3, _NC),
                  ((0, 0), (0, 2), (0, 7))).reshape(_HID, 1024)
    asf3 = jnp.pad(a_src3, ((0, 2), (0, 7))).reshape(1, 1024)
    adf3 = jnp.pad(a_dst3, ((0, 2), (0, 7))).reshape(1, 1024)
    hp3, as3, ad3, skip3 = _tc_mid(
        msg2.reshape(_NPAD, _HID), den2f.reshape(_NPAD, _H1), skip2,
        b2.reshape(1, _HID),
        W3p, Wl3, bl3.reshape(1, _NC), asf3, adf3, Ek, _H3, 128, _NC)
    msg3, den3f = _gat_sc(hp3, as3.reshape(-1), ad3.reshape(-1), srcp, dstp,
                          sched16, zeros16, _H3, 128, 16)
    return _tc_final(msg3.reshape(_NPAD, _HID), den3f.reshape(_NPAD, _H3),
                     skip3, b3.reshape(1, _NC))


# speedup vs baseline: 1.0146x; 1.0146x over previous
import jax
import jax.numpy as jnp
from jax import lax
from jax.experimental import pallas as pl
from jax.experimental.pallas import tpu as pltpu
from jax.experimental.pallas import tpu_sc as plsc

_N = 10000
_E = 320000
_F = 50
_H1 = 4
_C1 = 256
_HID = 1024
_H3 = 6
_NC = 121
_K = 128
_SUB = 16
_NPW = 64
_NWIN = 160
_NPAD = _NWIN * _NPW
_ET = _E + _N
_ETP = ((_ET + _K - 1) // _K) * _K
_NWORK = 32
_WPW = _NWIN // _NWORK
_MBLK = 256
_GRID = _NPAD // _MBLK
_NEG = -3.0e38

_f32 = jnp.float32
_i32 = jnp.int32


_SC_CACHE = {}


def _gat_sc(h, asrc_f, adst_f, srcp, dstp, sched_f, zeros_f, H, C, NPW):
    key = (H, C, NPW)
    if key not in _SC_CACHE:
        _SC_CACHE[key] = _build_sc(H, C, NPW)
    return _SC_CACHE[key](srcp, dstp, sched_f, asrc_f, adst_f, h, zeros_f)


def _build_sc(H, C, NPW):
    D = _HID
    nwin = _NPAD // NPW
    wpw = nwin // _NWORK
    grp = _K // 16
    mesh = plsc.VectorSubcoreMesh(core_axis_name="c", subcore_axis_name="s",
                                  num_cores=2, num_subcores=16)

    def body(src_hbm, dst_hbm, sched_hbm, asrc_hbm, adst_hbm, h_hbm, zero_hbm,
             msg_hbm, den_hbm,
             src_c, dst_c, asrc_t, adslab, emax, dens, wbuf,
             rows, tmpf, tmpi, schedrow, acc, sem0, sem1):
        wid = lax.axis_index("s") * 2 + lax.axis_index("c")
        iota = lax.iota(_i32, 16)
        zf = jnp.zeros((16,), _f32)
        ninf = jnp.full((16,), _NEG, _f32)
        pltpu.sync_copy(asrc_hbm, asrc_t)

        def wbody(t, wcarry):
            win = wid * wpw + t
            d0 = win * NPW
            pltpu.sync_copy(sched_hbm.at[pl.ds(win * 16, 16)], schedrow)
            sv = schedrow[...]
            e0 = sv[0]
            e1 = sv[1]
            c0 = sv[2]
            nch = sv[3]
            for i in range(NPW * H // 16):
                emax[pl.ds(i * 16, 16)] = ninf
                dens[pl.ds(i * 16, 16)] = zf
            pltpu.sync_copy(adst_hbm.at[pl.ds(d0 * H, NPW * H)], adslab)
            pltpu.sync_copy(zero_hbm, acc)

            def prelude(g):
                dstv = dst_c[pl.ds(g * 16, 16)]
                dl = jnp.clip(dstv - d0, 0, NPW - 1)
                tmpi[...] = dl
                pdl = [plsc.load_gather(tmpi, [jnp.maximum(iota - d, 0)])
                       for d in (1, 2, 4, 8)]
                ndl = plsc.load_gather(tmpi, [jnp.minimum(iota + 1, 15)])
                last = (dl != ndl) | (iota == 15)
                return dl, pdl, last

            def escore(g, dl, valid, hcol):
                srcv = src_c[pl.ds(g * 16, 16)]
                ash = plsc.load_gather(asrc_t, [srcv * H + hcol])
                adh = plsc.load_gather(adslab, [dl * H + hcol])
                s = ash + adh
                ev = jnp.maximum(s, 0.2 * s)
                return jnp.where(valid, ev, _NEG)

            def p1(ci, carry):
                cbase = (c0 + ci) * _K
                pltpu.sync_copy(src_hbm.at[pl.ds(cbase, _K)], src_c)
                pltpu.sync_copy(dst_hbm.at[pl.ds(cbase, _K)], dst_c)
                for g in range(grp):
                    dl, pdl, last = prelude(g)
                    eidx = cbase + g * 16 + iota
                    valid = (eidx >= e0) & (eidx < e1)
                    for hcol in range(H):
                        m = escore(g, dl, valid, hcol)
                        for d, p in zip((1, 2, 4, 8), pdl):
                            tmpf[...] = m
                            prev = plsc.load_gather(
                                tmpf, [jnp.maximum(iota - d, 0)])
                            m = jnp.maximum(m, jnp.where(dl == p, prev, _NEG))
                        fi = dl * H + hcol
                        cur = plsc.load_gather(emax, [fi])
                        plsc.store_scatter(emax, [fi], jnp.maximum(cur, m),
                                           mask=last)
                return carry

            lax.fori_loop(0, nch, p1, 0)

            def p2(ci, carry):
                cbase = (c0 + ci) * _K
                pltpu.sync_copy(src_hbm.at[pl.ds(cbase, _K)], src_c)
                pltpu.sync_copy(dst_hbm.at[pl.ds(cbase, _K)], dst_c)
                for g in range(grp):
                    dl, pdl, last = prelude(g)
                    eidx = cbase + g * 16 + iota
                    valid = (eidx >= e0) & (eidx < e1)
                    for hcol in range(H):
                        ev = escore(g, dl, valid, hcol)
                        fi = dl * H + hcol
                        mx = plsc.load_gather(emax, [fi])
                        w = jnp.exp(ev - mx)
                        w = jnp.where(valid, w, 0.0)
                        plsc.store_scatter(wbuf, [(g * 16 + iota) * H + hcol],
                                           w)
                        m = w
                        for d, p in zip((1, 2, 4, 8), pdl):
                            tmpf[...] = m
                            prev = plsc.load_gather(
                                tmpf, [jnp.maximum(iota - d, 0)])
                            m = m + jnp.where((dl == p) & (iota >= d),
                                              prev, 0.0)
                        cur = plsc.load_gather(dens, [fi])
                        plsc.store_scatter(dens, [fi], cur + m, mask=last)
                sems = (sem0, sem1)
                cps = {0: pltpu.async_copy(
                    h_hbm.at[src_c[pl.ds(0, 16)]], rows.at[0], sem0)}
                for s in range(grp):
                    rb = s % 2
                    cps[s].wait()
                    if s + 1 < grp:
                        nb = (s + 1) % 2
                        cps[s + 1] = pltpu.async_copy(
                            h_hbm.at[src_c[pl.ds((s + 1) * 16, 16)]],
                            rows.at[nb], sems[nb])

                    def jbody(j, c2, s=s, rb=rb):
                        je = s * 16 + j
                        djv = plsc.load_gather(
                            dst_c, [jnp.full((16,), je, _i32)])
                        dj = jnp.clip(djv - d0, 0, NPW - 1)
                        dbase = dj * D
                        frb = jnp.full((16,), rb, _i32)
                        fj = jnp.full((16,), j, _i32)
                        for hcol in range(H):
                            wsp = plsc.load_gather(
                                wbuf,
                                [jnp.full((16,), je * H + hcol, _i32)])
                            for kk in range(C // 16):
                                off = hcol * C + kk * 16
                                v = plsc.load_gather(
                                    rows, [frb, fj, off + iota])
                                plsc.addupdate_scatter(
                                    acc, [dbase + (off + iota)], wsp * v)
                        return c2

                    lax.fori_loop(0, 16, jbody, 0)
                return carry

            lax.fori_loop(0, nch, p2, 0)
            pltpu.sync_copy(acc, msg_hbm.at[pl.ds(d0 * D, NPW * D)])
            pltpu.sync_copy(dens, den_hbm.at[pl.ds(d0 * H, NPW * H)])
            return wcarry

        lax.fori_loop(0, wpw, wbody, 0)

    kfn = pl.kernel(
        body,
        out_type=(jax.ShapeDtypeStruct((_NPAD * D,), _f32),
                  jax.ShapeDtypeStruct((_NPAD * H,), _f32)),
        mesh=mesh,
        compiler_params=pltpu.CompilerParams(needs_layout_passes=False),
        scratch_types=[
            pltpu.VMEM((_K,), _i32),
            pltpu.VMEM((_K,), _i32),
            pltpu.VMEM((_NPAD * H,), _f32),
            pltpu.VMEM((NPW * H,), _f32),
            pltpu.VMEM((NPW * H,), _f32),
            pltpu.VMEM((NPW * H,), _f32),
            pltpu.VMEM((_K * H,), _f32),
            pltpu.VMEM((2, 16, D), _f32),
            pltpu.VMEM((16,), _f32),
            pltpu.VMEM((16,), _i32),
            pltpu.VMEM((16,), _i32),
            pltpu.VMEM((NPW * D,), _f32),
            pltpu.SemaphoreType.DMA,
            pltpu.SemaphoreType.DMA,
        ],
    )
    return kfn


def _alpha_cols(hp, aflat, H, C):
    cols = [jnp.sum(hp[:, hd * C:(hd + 1) * C] * aflat[:, hd * C:(hd + 1) * C],
                    axis=1, keepdims=True) for hd in range(H)]
    return jnp.concatenate(cols, axis=1)


def _tc_first(x, W, Wl, bl, asf, adf):
    def body(x_ref, w_ref, wl_ref, bl_ref, asf_ref, adf_ref,
             hp_ref, as_ref, ad_ref, skip_ref):
        hp = jnp.dot(x_ref[...], w_ref[...], preferred_element_type=_f32)
        hp_ref[...] = hp
        as_ref[...] = _alpha_cols(hp, asf_ref[...], _H1, _C1)
        ad_ref[...] = _alpha_cols(hp, adf_ref[...], _H1, _C1)
        skip_ref[...] = jnp.dot(x_ref[...], wl_ref[...],
                                preferred_element_type=_f32) + bl_ref[...]

    return pl.pallas_call(
        body,
        grid=(_GRID,),
        in_specs=[
            pl.BlockSpec((_MBLK, _F), lambda i: (i, 0)),
            pl.BlockSpec((_F, _HID), lambda i: (0, 0)),
            pl.BlockSpec((_F, _HID), lambda i: (0, 0)),
            pl.BlockSpec((1, _HID), lambda i: (0, 0)),
            pl.BlockSpec((1, _HID), lambda i: (0, 0)),
            pl.BlockSpec((1, _HID), lambda i: (0, 0)),
        ],
        out_specs=[
            pl.BlockSpec((_MBLK, _HID), lambda i: (i, 0)),
            pl.BlockSpec((_MBLK, _H1), lambda i: (i, 0)),
            pl.BlockSpec((_MBLK, _H1), lambda i: (i, 0)),
            pl.BlockSpec((_MBLK, _HID), lambda i: (i, 0)),
        ],
        out_shape=[
            jax.ShapeDtypeStruct((_N, _HID), _f32),
            jax.ShapeDtypeStruct((_NPAD, _H1), _f32),
            jax.ShapeDtypeStruct((_NPAD, _H1), _f32),
            jax.ShapeDtypeStruct((_N, _HID), _f32),
        ],
    )(x, W, Wl, bl, asf, adf)


def _tc_mid(msg, den, skipp, b_row, W, Wl, bl_row, asf, adf, Ek, H, C, Dsk):
    def body(msg_ref, den_ref, skipp_ref, b_ref, w_ref, wl_ref, bl_ref,
             asf_ref, adf_ref, ek_ref,
             hp_ref, as_ref, ad_ref, skip_ref):
        denr = jnp.dot(den_ref[...], ek_ref[...], preferred_element_type=_f32)
        z = msg_ref[...] / (denr + 1e-16) + b_ref[...] + skipp_ref[...]
        z = jnp.where(z > 0, z, jnp.exp(z) - 1.0)
        hp = jnp.dot(z, w_ref[...], preferred_element_type=_f32)
        hp_ref[...] = hp
        as_ref[...] = _alpha_cols(hp, asf_ref[...], H, C)
        ad_ref[...] = _alpha_cols(hp, adf_ref[...], H, C)
        skip_ref[...] = jnp.dot(z, wl_ref[...],
                                preferred_element_type=_f32) + bl_ref[...]

    return pl.pallas_call(
        body,
        grid=(_GRID,),
        in_specs=[
            pl.BlockSpec((_MBLK, _HID), lambda i: (i, 0)),
            pl.BlockSpec((_MBLK, _H1), lambda i: (i, 0)),
            pl.BlockSpec((_MBLK, _HID), lambda i: (i, 0)),
            pl.BlockSpec((1, _HID), lambda i: (0, 0)),
            pl.BlockSpec((_HID, _HID), lambda i: (0, 0)),
            pl.BlockSpec((_HID, Dsk), lambda i: (0, 0)),
            pl.BlockSpec((1, Dsk), lambda i: (0, 0)),
            pl.BlockSpec((1, _HID), lambda i: (0, 0)),
            pl.BlockSpec((1, _HID), lambda i: (0, 0)),
            pl.BlockSpec((_H1, _HID), lambda i: (0, 0)),
        ],
        out_specs=[
            pl.BlockSpec((_MBLK, _HID), lambda i: (i, 0)),
            pl.BlockSpec((_MBLK, H), lambda i: (i, 0)),
            pl.BlockSpec((_MBLK, H), lambda i: (i, 0)),
            pl.BlockSpec((_MBLK, Dsk), lambda i: (i, 0)),
        ],
        out_shape=[
            jax.ShapeDtypeStruct((_N, _HID), _f32),
            jax.ShapeDtypeStruct((_NPAD, H), _f32),
            jax.ShapeDtypeStruct((_NPAD, H), _f32),
            jax.ShapeDtypeStruct((_N, Dsk), _f32),
        ],
    )(msg, den, skipp, b_row, W, Wl, bl_row, asf, adf, Ek)


def _tc_final(msg3, den3, skip3, b3_row):
    def body(msg_ref, den_ref, skip_ref, b_ref, out_ref):
        acc = None
        for hd in range(_H3):
            term = (msg_ref[:, hd * 128:hd * 128 + _NC]
                    / (den_ref[:, hd:hd + 1] + 1e-16))
            acc = term if acc is None else acc + term
        out_ref[...] = acc * (1.0 / _H3) + b_ref[...] + skip_ref[...]

    return pl.pallas_call(
        body,
        grid=(_GRID,),
        in_specs=[
            pl.BlockSpec((_MBLK, _HID), lambda i: (i, 0)),
            pl.BlockSpec((_MBLK, _H3), lambda i: (i, 0)),
            pl.BlockSpec((_MBLK, _NC), lambda i: (i, 0)),
            pl.BlockSpec((1, _NC), lambda i: (0, 0)),
        ],
        out_specs=pl.BlockSpec((_MBLK, _NC), lambda i: (i, 0)),
        out_shape=jax.ShapeDtypeStruct((_N, _NC), _f32),
    )(msg3, den3, skip3, b3_row)


def _edge_prep(edge_index):
    loops = jnp.arange(_N, dtype=_i32)
    src0 = jnp.concatenate([edge_index[0], loops])
    dst0 = jnp.concatenate([edge_index[1], loops])
    dst_s, src_s = lax.sort((dst0, src0), num_keys=1)
    pad = _ETP - _ET
    srcp = jnp.concatenate([src_s, jnp.zeros((pad,), _i32)])
    dstp = jnp.concatenate([dst_s, jnp.full((pad,), _NPAD - 1, _i32)])
    return srcp, dstp


def _sched_for(dstp, npw):
    nwin = _NPAD // npw
    bnd = jnp.searchsorted(
        dstp, jnp.arange(0, _NPAD + 1, npw, dtype=_i32),
        side='left').astype(_i32)
    e0 = bnd[:-1]
    e1 = bnd[1:]
    c0 = e0 // _K
    nch = jnp.maximum((e1 + _K - 1) // _K - c0, 0)
    sched = jnp.concatenate(
        [e0[:, None], e1[:, None], c0[:, None], nch[:, None],
         jnp.zeros((nwin, 12), _i32)], axis=1)
    return sched.reshape(-1)


def kernel(x, edge_index, W1, a_src1, a_dst1, b1, Wl1, bl1, W2, a_src2,
           a_dst2, b2, Wl2, bl2, W3, a_src3, a_dst3, b3, Wl3, bl3):
    srcp, dstp = _edge_prep(edge_index)
    sched40 = _sched_for(dstp, 40)
    sched16 = _sched_for(dstp, 16)
    zeros40 = jnp.zeros((40 * _HID,), _f32)
    zeros16 = jnp.zeros((16 * _HID,), _f32)
    Ek = jnp.repeat(jnp.eye(_H1, dtype=_f32), _C1, axis=1)

    hp1, as1, ad1, skip1 = _tc_first(
        x, W1, Wl1, bl1.reshape(1, _HID), a_src1.reshape(1, _HID),
        a_dst1.reshape(1, _HID))
    msg1, den1f = _gat_sc(hp1, as1.reshape(-1), ad1.reshape(-1), srcp, dstp,
                          sched40, zeros40, _H1, _C1, 40)
    hp2, as2, ad2, skip2 = _tc_mid(
        msg1.reshape(_NPAD, _HID), den1f.reshape(_NPAD, _H1), skip1,
        b1.reshape(1, _HID),
        W2, Wl2, bl2.reshape(1, _HID), a_src2.reshape(1, _HID),
        a_dst2.reshape(1, _HID), Ek, _H1, _C1, _HID)
    msg2, den2f = _gat_sc(hp2, as2.reshape(-1), ad2.reshape(-1), srcp, dstp,
                          sched40, zeros40, _H1, _C1, 40)
    W3p = jnp.pad(W3.reshape(_HID, _H3, _NC),
                  ((0, 0), (0, 2), (0, 7))).reshape(_HID, 1024)
    asf3 = jnp.pad(a_src3, ((0, 2), (0, 7))).reshape(1, 1024)
    adf3 = jnp.pad(a_dst3, ((0, 2), (0, 7))).reshape(1, 1024)
    hp3, as3, ad3, skip3 = _tc_mid(
        msg2.reshape(_NPAD, _HID), den2f.reshape(_NPAD, _H1), skip2,
        b2.reshape(1, _HID),
        W3p, Wl3, bl3.reshape(1, _NC), asf3, adf3, Ek, _H3, 128, _NC)
    msg3, den3f = _gat_sc(hp3, as3.reshape(-1), ad3.reshape(-1), srcp, dstp,
                          sched16, zeros16, _H3, 128, 16)
    return _tc_final(msg3.reshape(_NPAD, _HID), den3f.reshape(_NPAD, _H3),
                     skip3, b3.reshape(1, _NC))


# speedup vs baseline: 1.0738x; 1.0584x over previous
import jax
import jax.numpy as jnp
from jax import lax
from jax.experimental import pallas as pl
from jax.experimental.pallas import tpu as pltpu
from jax.experimental.pallas import tpu_sc as plsc

_N = 10000
_E = 320000
_F = 50
_H1 = 4
_C1 = 256
_HID = 1024
_H3 = 6
_NC = 121
_K = 128
_SUB = 16
_NPW = 64
_NWIN = 160
_NPAD = _NWIN * _NPW
_ET = _E + _N
_ETP = ((_ET + _K - 1) // _K) * _K
_NWORK = 32
_WPW = _NWIN // _NWORK
_MBLK = 256
_GRID = _NPAD // _MBLK
_NEG = -3.0e38

_f32 = jnp.float32
_i32 = jnp.int32


_SC_CACHE = {}


def _gat_sc(h, asrc_f, adst_f, srcp, dstp, sched_f, zeros_f, H, C, NPW):
    key = (H, C, NPW)
    if key not in _SC_CACHE:
        _SC_CACHE[key] = _build_sc(H, C, NPW)
    return _SC_CACHE[key](srcp, dstp, sched_f, asrc_f, adst_f, h, zeros_f)


def _build_sc(H, C, NPW):
    D = _HID
    nwin = _NPAD // NPW
    wpw = nwin // _NWORK
    grp = _K // 16
    mesh = plsc.VectorSubcoreMesh(core_axis_name="c", subcore_axis_name="s",
                                  num_cores=2, num_subcores=16)

    def body(src_hbm, dst_hbm, sched_hbm, asrc_hbm, adst_hbm, h_hbm, zero_hbm,
             msg_hbm, den_hbm,
             src_c, dst_c, asrc_t, adslab, emax, dens, wbuf,
             rows, tmpf, tmpi, schedrow, acc, sem0):
        wid = lax.axis_index("s") * 2 + lax.axis_index("c")
        iota = lax.iota(_i32, 16)
        zf = jnp.zeros((16,), _f32)
        ninf = jnp.full((16,), _NEG, _f32)
        pltpu.sync_copy(asrc_hbm, asrc_t)

        def wbody(t, wcarry):
            win = wid * wpw + t
            d0 = win * NPW
            pltpu.sync_copy(sched_hbm.at[pl.ds(win * 16, 16)], schedrow)
            sv = schedrow[...]
            e0 = sv[0]
            e1 = sv[1]
            c0 = sv[2]
            nch = sv[3]
            for i in range(NPW * H // 16):
                emax[pl.ds(i * 16, 16)] = ninf
                dens[pl.ds(i * 16, 16)] = zf
            pltpu.sync_copy(adst_hbm.at[pl.ds(d0 * H, NPW * H)], adslab)
            pltpu.sync_copy(zero_hbm, acc)

            def prelude(g):
                dstv = dst_c[pl.ds(g * 16, 16)]
                dl = jnp.clip(dstv - d0, 0, NPW - 1)
                tmpi[...] = dl
                pdl = [plsc.load_gather(tmpi, [jnp.maximum(iota - d, 0)])
                       for d in (1, 2, 4, 8)]
                ndl = plsc.load_gather(tmpi, [jnp.minimum(iota + 1, 15)])
                last = (dl != ndl) | (iota == 15)
                return dl, pdl, last

            def escore(g, dl, valid, hcol):
                srcv = src_c[pl.ds(g * 16, 16)]
                ash = plsc.load_gather(asrc_t, [srcv * H + hcol])
                adh = plsc.load_gather(adslab, [dl * H + hcol])
                s = ash + adh
                ev = jnp.maximum(s, 0.2 * s)
                return jnp.where(valid, ev, _NEG)

            def p1(ci, carry):
                cbase = (c0 + ci) * _K
                pltpu.sync_copy(src_hbm.at[pl.ds(cbase, _K)], src_c)
                pltpu.sync_copy(dst_hbm.at[pl.ds(cbase, _K)], dst_c)
                for g in range(grp):
                    dl, pdl, last = prelude(g)
                    eidx = cbase + g * 16 + iota
                    valid = (eidx >= e0) & (eidx < e1)
                    for hcol in range(H):
                        m = escore(g, dl, valid, hcol)
                        for d, p in zip((1, 2, 4, 8), pdl):
                            tmpf[...] = m
                            prev = plsc.load_gather(
                                tmpf, [jnp.maximum(iota - d, 0)])
                            m = jnp.maximum(m, jnp.where(dl == p, prev, _NEG))
                        fi = dl * H + hcol
                        cur = plsc.load_gather(emax, [fi])
                        plsc.store_scatter(emax, [fi], jnp.maximum(cur, m),
                                           mask=last)
                return carry

            lax.fori_loop(0, nch, p1, 0)

            def p2(ci, carry):
                cbase = (c0 + ci) * _K
                pltpu.sync_copy(src_hbm.at[pl.ds(cbase, _K)], src_c)
                pltpu.sync_copy(dst_hbm.at[pl.ds(cbase, _K)], dst_c)
                for g in range(grp):
                    dl, pdl, last = prelude(g)
                    eidx = cbase + g * 16 + iota
                    valid = (eidx >= e0) & (eidx < e1)
                    for hcol in range(H):
                        ev = escore(g, dl, valid, hcol)
                        fi = dl * H + hcol
                        mx = plsc.load_gather(emax, [fi])
                        w = jnp.exp(ev - mx)
                        w = jnp.where(valid, w, 0.0)
                        plsc.store_scatter(wbuf, [(g * 16 + iota) * H + hcol],
                                           w)
                        m = w
                        for d, p in zip((1, 2, 4, 8), pdl):
                            tmpf[...] = m
                            prev = plsc.load_gather(
                                tmpf, [jnp.maximum(iota - d, 0)])
                            m = m + jnp.where((dl == p) & (iota >= d),
                                              prev, 0.0)
                        cur = plsc.load_gather(dens, [fi])
                        plsc.store_scatter(dens, [fi], cur + m, mask=last)
                def sbody(s, sc2):
                    idxv = src_c[pl.ds(s * 16, 16)]
                    pltpu.async_copy(h_hbm.at[idxv], rows, sem0).wait()

                    def jbody(j, c2):
                        je = s * 16 + j
                        djv = plsc.load_gather(
                            dst_c, [jnp.full((16,), je, _i32)])
                        dj = jnp.clip(djv - d0, 0, NPW - 1)
                        dbase = dj * D
                        fj = jnp.full((16,), j, _i32)
                        for hcol in range(H):
                            wsp = plsc.load_gather(
                                wbuf,
                                [jnp.full((16,), je * H + hcol, _i32)])
                            for kk in range(C // 16):
                                off = hcol * C + kk * 16
                                v = plsc.load_gather(rows, [fj, off + iota])
                                plsc.addupdate_scatter(
                                    acc, [dbase + (off + iota)], wsp * v)
                        return c2

                    lax.fori_loop(0, 16, jbody, 0)
                    return sc2

                lax.fori_loop(0, grp, sbody, 0)
                return carry

            lax.fori_loop(0, nch, p2, 0)
            pltpu.sync_copy(acc, msg_hbm.at[pl.ds(d0 * D, NPW * D)])
            pltpu.sync_copy(dens, den_hbm.at[pl.ds(d0 * H, NPW * H)])
            return wcarry

        lax.fori_loop(0, wpw, wbody, 0)

    kfn = pl.kernel(
        body,
        out_type=(jax.ShapeDtypeStruct((_NPAD * D,), _f32),
                  jax.ShapeDtypeStruct((_NPAD * H,), _f32)),
        mesh=mesh,
        compiler_params=pltpu.CompilerParams(needs_layout_passes=False),
        scratch_types=[
            pltpu.VMEM((_K,), _i32),
            pltpu.VMEM((_K,), _i32),
            pltpu.VMEM((_NPAD * H,), _f32),
            pltpu.VMEM((NPW * H,), _f32),
            pltpu.VMEM((NPW * H,), _f32),
            pltpu.VMEM((NPW * H,), _f32),
            pltpu.VMEM((_K * H,), _f32),
            pltpu.VMEM((16, D), _f32),
            pltpu.VMEM((16,), _f32),
            pltpu.VMEM((16,), _i32),
            pltpu.VMEM((16,), _i32),
            pltpu.VMEM((NPW * D,), _f32),
            pltpu.SemaphoreType.DMA,
        ],
    )
    return kfn


def _alpha_cols(hp, aflat, H, C):
    cols = [jnp.sum(hp[:, hd * C:(hd + 1) * C] * aflat[:, hd * C:(hd + 1) * C],
                    axis=1, keepdims=True) for hd in range(H)]
    return jnp.concatenate(cols, axis=1)


def _tc_first(x, W, Wl, bl, asf, adf):
    def body(x_ref, w_ref, wl_ref, bl_ref, asf_ref, adf_ref,
             hp_ref, as_ref, ad_ref, skip_ref):
        hp = jnp.dot(x_ref[...], w_ref[...], preferred_element_type=_f32)
        hp_ref[...] = hp
        as_ref[...] = _alpha_cols(hp, asf_ref[...], _H1, _C1)
        ad_ref[...] = _alpha_cols(hp, adf_ref[...], _H1, _C1)
        skip_ref[...] = jnp.dot(x_ref[...], wl_ref[...],
                                preferred_element_type=_f32) + bl_ref[...]

    return pl.pallas_call(
        body,
        grid=(_GRID,),
        in_specs=[
            pl.BlockSpec((_MBLK, _F), lambda i: (i, 0)),
            pl.BlockSpec((_F, _HID), lambda i: (0, 0)),
            pl.BlockSpec((_F, _HID), lambda i: (0, 0)),
            pl.BlockSpec((1, _HID), lambda i: (0, 0)),
            pl.BlockSpec((1, _HID), lambda i: (0, 0)),
            pl.BlockSpec((1, _HID), lambda i: (0, 0)),
        ],
        out_specs=[
            pl.BlockSpec((_MBLK, _HID), lambda i: (i, 0)),
            pl.BlockSpec((_MBLK, _H1), lambda i: (i, 0)),
            pl.BlockSpec((_MBLK, _H1), lambda i: (i, 0)),
            pl.BlockSpec((_MBLK, _HID), lambda i: (i, 0)),
        ],
        out_shape=[
            jax.ShapeDtypeStruct((_N, _HID), _f32),
            jax.ShapeDtypeStruct((_NPAD, _H1), _f32),
            jax.ShapeDtypeStruct((_NPAD, _H1), _f32),
            jax.ShapeDtypeStruct((_N, _HID), _f32),
        ],
    )(x, W, Wl, bl, asf, adf)


def _tc_mid(msg, den, skipp, b_row, W, Wl, bl_row, asf, adf, Ek, H, C, Dsk):
    def body(msg_ref, den_ref, skipp_ref, b_ref, w_ref, wl_ref, bl_ref,
             asf_ref, adf_ref, ek_ref,
             hp_ref, as_ref, ad_ref, skip_ref):
        denr = jnp.dot(den_ref[...], ek_ref[...], preferred_element_type=_f32)
        z = msg_ref[...] / (denr + 1e-16) + b_ref[...] + skipp_ref[...]
        z = jnp.where(z > 0, z, jnp.exp(z) - 1.0)
        hp = jnp.dot(z, w_ref[...], preferred_element_type=_f32)
        hp_ref[...] = hp
        as_ref[...] = _alpha_cols(hp, asf_ref[...], H, C)
        ad_ref[...] = _alpha_cols(hp, adf_ref[...], H, C)
        skip_ref[...] = jnp.dot(z, wl_ref[...],
                                preferred_element_type=_f32) + bl_ref[...]

    return pl.pallas_call(
        body,
        grid=(_GRID,),
        in_specs=[
            pl.BlockSpec((_MBLK, _HID), lambda i: (i, 0)),
            pl.BlockSpec((_MBLK, _H1), lambda i: (i, 0)),
            pl.BlockSpec((_MBLK, _HID), lambda i: (i, 0)),
            pl.BlockSpec((1, _HID), lambda i: (0, 0)),
            pl.BlockSpec((_HID, _HID), lambda i: (0, 0)),
            pl.BlockSpec((_HID, Dsk), lambda i: (0, 0)),
            pl.BlockSpec((1, Dsk), lambda i: (0, 0)),
            pl.BlockSpec((1, _HID), lambda i: (0, 0)),
            pl.BlockSpec((1, _HID), lambda i: (0, 0)),
            pl.BlockSpec((_H1, _HID), lambda i: (0, 0)),
        ],
        out_specs=[
            pl.BlockSpec((_MBLK, _HID), lambda i: (i, 0)),
            pl.BlockSpec((_MBLK, H), lambda i: (i, 0)),
            pl.BlockSpec((_MBLK, H), lambda i: (i, 0)),
            pl.BlockSpec((_MBLK, Dsk), lambda i: (i, 0)),
        ],
        out_shape=[
            jax.ShapeDtypeStruct((_N, _HID), _f32),
            jax.ShapeDtypeStruct((_NPAD, H), _f32),
            jax.ShapeDtypeStruct((_NPAD, H), _f32),
            jax.ShapeDtypeStruct((_N, Dsk), _f32),
        ],
    )(msg, den, skipp, b_row, W, Wl, bl_row, asf, adf, Ek)


def _tc_final(msg3, den3, skip3, b3_row):
    def body(msg_ref, den_ref, skip_ref, b_ref, out_ref):
        acc = None
        for hd in range(_H3):
            term = (msg_ref[:, hd * 128:hd * 128 + _NC]
                    / (den_ref[:, hd:hd + 1] + 1e-16))
            acc = term if acc is None else acc + term
        out_ref[...] = acc * (1.0 / _H3) + b_ref[...] + skip_ref[...]

    return pl.pallas_call(
        body,
        grid=(_GRID,),
        in_specs=[
            pl.BlockSpec((_MBLK, _HID), lambda i: (i, 0)),
            pl.BlockSpec((_MBLK, _H3), lambda i: (i, 0)),
            pl.BlockSpec((_MBLK, _NC), lambda i: (i, 0)),
            pl.BlockSpec((1, _NC), lambda i: (0, 0)),
        ],
        out_specs=pl.BlockSpec((_MBLK, _NC), lambda i: (i, 0)),
        out_shape=jax.ShapeDtypeStruct((_N, _NC), _f32),
    )(msg3, den3, skip3, b3_row)


def _edge_prep(edge_index):
    loops = jnp.arange(_N, dtype=_i32)
    src0 = jnp.concatenate([edge_index[0], loops])
    dst0 = jnp.concatenate([edge_index[1], loops])
    dst_s, src_s = lax.sort((dst0, src0), num_keys=1)
    pad = _ETP - _ET
    srcp = jnp.concatenate([src_s, jnp.zeros((pad,), _i32)])
    dstp = jnp.concatenate([dst_s, jnp.full((pad,), _NPAD - 1, _i32)])
    return srcp, dstp


def _sched_for(dstp, npw):
    nwin = _NPAD // npw
    bnd = jnp.searchsorted(
        dstp, jnp.arange(0, _NPAD + 1, npw, dtype=_i32),
        side='left').astype(_i32)
    e0 = bnd[:-1]
    e1 = bnd[1:]
    c0 = e0 // _K
    nch = jnp.maximum((e1 + _K - 1) // _K - c0, 0)
    sched = jnp.concatenate(
        [e0[:, None], e1[:, None], c0[:, None], nch[:, None],
         jnp.zeros((nwin, 12), _i32)], axis=1)
    return sched.reshape(-1)


def kernel(x, edge_index, W1, a_src1, a_dst1, b1, Wl1, bl1, W2, a_src2,
           a_dst2, b2, Wl2, bl2, W3, a_src3, a_dst3, b3, Wl3, bl3):
    srcp, dstp = _edge_prep(edge_index)
    sched64 = _sched_for(dstp, 64)
    sched32 = _sched_for(dstp, 32)
    zeros64 = jnp.zeros((64 * _HID,), _f32)
    zeros32 = jnp.zeros((32 * _HID,), _f32)
    Ek = jnp.repeat(jnp.eye(_H1, dtype=_f32), _C1, axis=1)

    hp1, as1, ad1, skip1 = _tc_first(
        x, W1, Wl1, bl1.reshape(1, _HID), a_src1.reshape(1, _HID),
        a_dst1.reshape(1, _HID))
    msg1, den1f = _gat_sc(hp1, as1.reshape(-1), ad1.reshape(-1), srcp, dstp,
                          sched64, zeros64, _H1, _C1, 64)
    hp2, as2, ad2, skip2 = _tc_mid(
        msg1.reshape(_NPAD, _HID), den1f.reshape(_NPAD, _H1), skip1,
        b1.reshape(1, _HID),
        W2, Wl2, bl2.reshape(1, _HID), a_src2.reshape(1, _HID),
        a_dst2.reshape(1, _HID), Ek, _H1, _C1, _HID)
    msg2, den2f = _gat_sc(hp2, as2.reshape(-1), ad2.reshape(-1), srcp, dstp,
                          sched64, zeros64, _H1, _C1, 64)
    W3p = jnp.pad(W3.reshape(_HID, _H3, _NC),
                  ((0, 0), (0, 2), (0, 7))).reshape(_HID, 1024)
    asf3 = jnp.pad(a_src3, ((0, 2), (0, 7))).reshape(1, 1024)
    adf3 = jnp.pad(a_dst3, ((0, 2), (0, 7))).reshape(1, 1024)
    hp3, as3, ad3, skip3 = _tc_mid(
        msg2.reshape(_NPAD, _HID), den2f.reshape(_NPAD, _H1), skip2,
        b2.reshape(1, _HID),
        W3p, Wl3, bl3.reshape(1, _NC), asf3, adf3, Ek, _H3, 128, _NC)
    msg3, den3f = _gat_sc(hp3, as3.reshape(-1), ad3.reshape(-1), srcp, dstp,
                          sched32, zeros32, _H3, 128, 32)
    return _tc_final(msg3.reshape(_NPAD, _HID), den3f.reshape(_NPAD, _H3),
                     skip3, b3.reshape(1, _NC))
